# Initial kernel scaffold; baseline (speedup 1.0000x reference)
import jax
import jax.numpy as jnp
from jax.experimental import pallas as pl


def kernel(x, splits, thresholds, values):
    raise NotImplementedError("write your pallas kernel here")


# speedup vs baseline: 58.1271x; 58.1271x over previous
import functools

import jax
import jax.numpy as jnp
import numpy as np
from jax.experimental import pallas as pl

NUM_TREES = 10
MAX_DEPTH = 3
NUM_LEAVES = 2 ** MAX_DEPTH
BLOCK_ROWS = 2000


def _forest_kernel(x_ref, w_ref, thr_ref, code_w_ref, vals_ref, out_ref):
    xb = x_ref[:]
    proj = jnp.dot(xb, w_ref[:], preferred_element_type=jnp.float32)
    bits = (proj > thr_ref[:]).astype(jnp.float32)
    codes = jnp.dot(bits, code_w_ref[:], preferred_element_type=jnp.float32)
    acc = jnp.zeros(codes.shape, dtype=jnp.float32)
    for k in range(NUM_LEAVES):
        acc += jnp.where(codes == float(k), vals_ref[k, :][None, :], 0.0)
    out_ref[:] = jnp.sum(acc, axis=1, keepdims=True)


def kernel(x, splits, thresholds, values):
    n, d = x.shape
    t, depth = thresholds.shape
    w = splits.reshape(t * depth, d).T
    thr = thresholds.reshape(1, t * depth)
    bitw = (2.0 ** np.arange(depth - 1, -1, -1)).astype(np.float32)
    code_w = np.zeros((t * depth, t), dtype=np.float32)
    for ti in range(t):
        code_w[ti * depth:(ti + 1) * depth, ti] = bitw
    code_w = jnp.asarray(code_w)
    vals = (jnp.roll(values, 1, axis=1) / t).T

    grid = (n // BLOCK_ROWS,)
    return pl.pallas_call(
        _forest_kernel,
        grid=grid,
        in_specs=[
            pl.BlockSpec((BLOCK_ROWS, d), lambda i: (i, 0)),
            pl.BlockSpec((d, t * depth), lambda i: (0, 0)),
            pl.BlockSpec((1, t * depth), lambda i: (0, 0)),
            pl.BlockSpec((t * depth, t), lambda i: (0, 0)),
            pl.BlockSpec((NUM_LEAVES, t), lambda i: (0, 0)),
        ],
        out_specs=pl.BlockSpec((BLOCK_ROWS, 1), lambda i: (i, 0)),
        out_shape=jax.ShapeDtypeStruct((n, 1), jnp.float32),
    )(x, w, thr, code_w, vals)


# speedup vs baseline: 70.5730x; 1.2141x over previous
import functools

import jax
import jax.numpy as jnp
import numpy as np
from jax.experimental import pallas as pl

NUM_TREES = 10
MAX_DEPTH = 3
NUM_LEAVES = 2 ** MAX_DEPTH
BLOCK_ROWS = 5000


def _forest_kernel(x_ref, w_ref, thr_ref, code_w_ref, vals_ref, out_ref):
    xb = x_ref[:]
    proj = jnp.dot(xb, w_ref[:], preferred_element_type=jnp.float32)
    bits = (proj > thr_ref[:]).astype(jnp.float32)
    codes = jnp.dot(bits, code_w_ref[:], preferred_element_type=jnp.float32)
    acc = jnp.zeros(codes.shape, dtype=jnp.float32)
    for k in range(NUM_LEAVES):
        acc += jnp.where(codes == float(k), vals_ref[k, :][None, :], 0.0)
    out_ref[:] = jnp.sum(acc, axis=1, keepdims=True)


def kernel(x, splits, thresholds, values):
    n, d = x.shape
    t, depth = thresholds.shape
    w = splits.reshape(t * depth, d).T
    thr = thresholds.reshape(1, t * depth)
    bitw = (2.0 ** np.arange(depth - 1, -1, -1)).astype(np.float32)
    code_w = np.zeros((t * depth, t), dtype=np.float32)
    for ti in range(t):
        code_w[ti * depth:(ti + 1) * depth, ti] = bitw
    code_w = jnp.asarray(code_w)
    vals = (jnp.roll(values, 1, axis=1) / t).T

    grid = (n // BLOCK_ROWS,)
    return pl.pallas_call(
        _forest_kernel,
        grid=grid,
        in_specs=[
            pl.BlockSpec((BLOCK_ROWS, d), lambda i: (i, 0)),
            pl.BlockSpec((d, t * depth), lambda i: (0, 0)),
            pl.BlockSpec((1, t * depth), lambda i: (0, 0)),
            pl.BlockSpec((t * depth, t), lambda i: (0, 0)),
            pl.BlockSpec((NUM_LEAVES, t), lambda i: (0, 0)),
        ],
        out_specs=pl.BlockSpec((BLOCK_ROWS, 1), lambda i: (i, 0)),
        out_shape=jax.ShapeDtypeStruct((n, 1), jnp.float32),
    )(x, w, thr, code_w, vals)


# speedup vs baseline: 74.5723x; 1.0567x over previous
import functools

import jax
import jax.numpy as jnp
import numpy as np
from jax.experimental import pallas as pl

NUM_TREES = 10
MAX_DEPTH = 3
NUM_LEAVES = 2 ** MAX_DEPTH
BLOCK_ROWS = 10000


def _forest_kernel(x_ref, w_ref, thr_ref, code_w_ref, vals_ref, out_ref):
    xb = x_ref[:]
    proj = jnp.dot(xb, w_ref[:], preferred_element_type=jnp.float32)
    bits = (proj > thr_ref[:]).astype(jnp.float32)
    codes = jnp.dot(bits, code_w_ref[:], preferred_element_type=jnp.float32)
    acc = jnp.zeros(codes.shape, dtype=jnp.float32)
    for k in range(NUM_LEAVES):
        acc += jnp.where(codes == float(k), vals_ref[k, :][None, :], 0.0)
    out_ref[:] = jnp.sum(acc, axis=1, keepdims=True)


def kernel(x, splits, thresholds, values):
    n, d = x.shape
    t, depth = thresholds.shape
    w = splits.reshape(t * depth, d).T
    thr = thresholds.reshape(1, t * depth)
    bitw = (2.0 ** np.arange(depth - 1, -1, -1)).astype(np.float32)
    code_w = np.zeros((t * depth, t), dtype=np.float32)
    for ti in range(t):
        code_w[ti * depth:(ti + 1) * depth, ti] = bitw
    code_w = jnp.asarray(code_w)
    vals = (jnp.roll(values, 1, axis=1) / t).T

    grid = (n // BLOCK_ROWS,)
    return pl.pallas_call(
        _forest_kernel,
        grid=grid,
        in_specs=[
            pl.BlockSpec((BLOCK_ROWS, d), lambda i: (i, 0)),
            pl.BlockSpec((d, t * depth), lambda i: (0, 0)),
            pl.BlockSpec((1, t * depth), lambda i: (0, 0)),
            pl.BlockSpec((t * depth, t), lambda i: (0, 0)),
            pl.BlockSpec((NUM_LEAVES, t), lambda i: (0, 0)),
        ],
        out_specs=pl.BlockSpec((BLOCK_ROWS, 1), lambda i: (i, 0)),
        out_shape=jax.ShapeDtypeStruct((n, 1), jnp.float32),
    )(x, w, thr, code_w, vals)


# speedup vs baseline: 85.3351x; 1.1443x over previous
import jax
import jax.numpy as jnp
import numpy as np
from jax.experimental import pallas as pl

BLOCK_ROWS = 10000


def _forest_kernel(x_ref, w_ref, thr_ref, cr_ref, kpat_ref, vflat_ref, out_ref):
    proj = jnp.dot(x_ref[:], w_ref[:], preferred_element_type=jnp.float32)
    bits = (proj > thr_ref[:]).astype(jnp.float32)
    rep = jnp.dot(bits, cr_ref[:], preferred_element_type=jnp.float32)
    oh = (rep == kpat_ref[:]).astype(jnp.float32)
    out_ref[:] = jnp.dot(oh, vflat_ref[:], preferred_element_type=jnp.float32)


def kernel(x, splits, thresholds, values):
    n, d = x.shape
    t, depth = thresholds.shape
    leaves = values.shape[1]
    w = splits.reshape(t * depth, d).T
    thr = thresholds.reshape(1, t * depth)
    cr = np.zeros((t * depth, t * leaves), dtype=np.float32)
    for ti in range(t):
        for dd in range(depth):
            cr[ti * depth + dd, ti * leaves:(ti + 1) * leaves] = 2.0 ** (depth - 1 - dd)
    cr = jnp.asarray(cr)
    kpat = jnp.asarray(np.tile(np.arange(leaves, dtype=np.float32), t)[None, :])
    vflat = (jnp.roll(values, 1, axis=1) / t).reshape(t * leaves, 1)

    grid = (n // BLOCK_ROWS,)
    return pl.pallas_call(
        _forest_kernel,
        grid=grid,
        in_specs=[
            pl.BlockSpec((BLOCK_ROWS, d), lambda i: (i, 0)),
            pl.BlockSpec((d, t * depth), lambda i: (0, 0)),
            pl.BlockSpec((1, t * depth), lambda i: (0, 0)),
            pl.BlockSpec((t * depth, t * leaves), lambda i: (0, 0)),
            pl.BlockSpec((1, t * leaves), lambda i: (0, 0)),
            pl.BlockSpec((t * leaves, 1), lambda i: (0, 0)),
        ],
        out_specs=pl.BlockSpec((BLOCK_ROWS, 1), lambda i: (i, 0)),
        out_shape=jax.ShapeDtypeStruct((n, 1), jnp.float32),
    )(x, w, thr, cr, kpat, vflat)
